# baseline (speedup 1.0000x reference)
import functools

import jax
import jax.numpy as jnp
from jax import lax
from jax.experimental import pallas as pl
from jax.experimental.pallas import tpu as pltpu
from jax.experimental.pallas import tpu_sc as plsc

VOCAB = 1000000
EMB = 32
SIDE = 32
BATCH = 16384

PACK = 4
NLINES = VOCAB // PACK
LINE = PACK * EMB

NUM_CORES = 2
NUM_SUBCORES = 16
NW = NUM_CORES * NUM_SUBCORES
B_PER_W = BATCH // NW
CHUNK = 128
NCHUNK = B_PER_W // CHUNK


def _sc_gather(table, gid):
    mesh = plsc.VectorSubcoreMesh(core_axis_name="c", subcore_axis_name="s")

    @functools.partial(
        pl.kernel,
        mesh=mesh,
        out_type=jax.ShapeDtypeStruct((BATCH, LINE), jnp.float32),
        scratch_types=[
            pltpu.VMEM((NCHUNK, CHUNK), jnp.int32),
            pltpu.VMEM((NCHUNK, CHUNK, LINE), jnp.float32),
            pltpu.SemaphoreType.DMA,
        ],
    )
    def k(table_hbm, gid_hbm, out_hbm, idx_v, rows_v, sem):
        wid = lax.axis_index("s") * NUM_CORES + lax.axis_index("c")
        base = wid * B_PER_W
        for j in range(NCHUNK):
            pltpu.sync_copy(
                gid_hbm.at[pl.ds(base + j * CHUNK, CHUNK)], idx_v.at[j]
            )
        copies = []
        for j in range(NCHUNK):
            copies.append(
                pltpu.async_copy(table_hbm.at[idx_v.at[j]], rows_v.at[j], sem)
            )
        for c in copies:
            c.wait()
        for j in range(NCHUNK):
            pltpu.sync_copy(
                rows_v.at[j], out_hbm.at[pl.ds(base + j * CHUNK, CHUNK)]
            )

    return k(table, gid)


REPACK_BLK = 2048
REPACK_GRID = 128
NLINES_PAD = REPACK_GRID * REPACK_BLK
_LAST_SAFE_BLK = VOCAB // REPACK_BLK - 1
_TAIL_TILE0 = 7808
_TAIL_OUT_BLK = (999424 - 3 * NLINES_PAD) // REPACK_BLK
_TAIL_ROWS = 5 * 128


def _repack_body(t0_ref, t1_ref, t2_ref, t3_ref, u0_ref, u1_ref, u2_ref,
                 u3_ref, u4_ref, eye_ref, out_ref):
    cat = jnp.concatenate(
        [t0_ref[...], t1_ref[...], t2_ref[...], t3_ref[...]], axis=0
    )
    out_ref[...] = lax.dot_general(
        cat, eye_ref[...], (((0,), (0,)), ((), ())),
        preferred_element_type=jnp.float32,
    )

    @pl.when(pl.program_id(0) == _TAIL_OUT_BLK)
    def _fix_tail():
        tail = jnp.concatenate(
            [u0_ref[...], u1_ref[...], u2_ref[...], u3_ref[...], u4_ref[...]],
            axis=1,
        )
        tail_t = lax.dot_general(
            tail, eye_ref[0:EMB, 0:EMB], (((0,), (0,)), ((), ())),
            preferred_element_type=jnp.float32,
        )
        out_ref[0:_TAIL_ROWS, 3 * EMB : 4 * EMB] = tail_t


def _repack(table_t):
    specs = [
        pl.BlockSpec(
            (EMB, REPACK_BLK),
            functools.partial(
                lambda a, i: (0, jnp.minimum(i + a * REPACK_GRID, _LAST_SAFE_BLK)),
                a,
            ),
        )
        for a in range(PACK)
    ]
    specs += [
        pl.BlockSpec(
            (EMB, 128), functools.partial(lambda k, i: (0, _TAIL_TILE0 + k), k)
        )
        for k in range(5)
    ]
    specs.append(pl.BlockSpec((LINE, LINE), lambda i: (0, 0)))
    eye = jnp.eye(LINE, dtype=jnp.float32)
    return pl.pallas_call(
        _repack_body,
        grid=(REPACK_GRID,),
        in_specs=specs,
        out_specs=pl.BlockSpec((REPACK_BLK, LINE), lambda i: (i, 0)),
        out_shape=jax.ShapeDtypeStruct((NLINES_PAD, LINE), jnp.float32),
        compiler_params=pltpu.CompilerParams(fuse_transposed_lhs_in_matmul=True),
    )(*([table_t] * 9), eye)


TC_BLK = 2048


def _tc_body(g_ref, oh_ref, side_ref, ws_ref, bs_ref, wl_ref, bl_ref, out_ref):
    g = g_ref[...]
    oh = oh_ref[...]
    emb = jnp.where(oh[:, 0:1] > 0.5, g[:, 0:EMB], 0.0)
    for r in range(1, PACK):
        emb += jnp.where(oh[:, r : r + 1] > 0.5, g[:, r * EMB : (r + 1) * EMB], 0.0)
    side = side_ref[...]
    side_emb = (
        jnp.dot(side, ws_ref[...], preferred_element_type=jnp.float32)
        + bs_ref[...]
    )
    out = (
        jnp.dot(emb, wl_ref[0:EMB, :], preferred_element_type=jnp.float32)
        + jnp.dot(side_emb, wl_ref[EMB:, :], preferred_element_type=jnp.float32)
        + bl_ref[...]
    )
    out_ref[...] = out


def _tc_dense(g, onehot, side_info, W_side, b_side, W_lin, b_lin):
    grid = (BATCH // TC_BLK,)
    return pl.pallas_call(
        _tc_body,
        grid=grid,
        in_specs=[
            pl.BlockSpec((TC_BLK, LINE), lambda i: (i, 0)),
            pl.BlockSpec((TC_BLK, PACK), lambda i: (i, 0)),
            pl.BlockSpec((TC_BLK, SIDE), lambda i: (i, 0)),
            pl.BlockSpec((SIDE, EMB), lambda i: (0, 0)),
            pl.BlockSpec((EMB,), lambda i: (0,)),
            pl.BlockSpec((2 * EMB, EMB), lambda i: (0, 0)),
            pl.BlockSpec((EMB,), lambda i: (0,)),
        ],
        out_specs=pl.BlockSpec((TC_BLK, EMB), lambda i: (i, 0)),
        out_shape=jax.ShapeDtypeStruct((BATCH, EMB), jnp.float32),
    )(g, onehot, side_info, W_side, b_side, W_lin, b_lin)


@jax.jit
def kernel(target, side_info, emb_table, W_side, b_side, W_lin, b_lin):
    idx = target.astype(jnp.int32)
    gid = jnp.bitwise_and(idx, NLINES_PAD - 1)
    slot = lax.shift_right_logical(idx, 18)
    onehot = (
        slot[:, None] == jnp.arange(PACK, dtype=jnp.int32)[None, :]
    ).astype(jnp.float32)
    table = _repack(emb_table.T)
    g = _sc_gather(table, gid)
    return _tc_dense(g, onehot, side_info, W_side, b_side, W_lin, b_lin)


# speedup vs baseline: 1.1462x; 1.1462x over previous
import functools

import jax
import jax.numpy as jnp
from jax import lax
from jax.experimental import pallas as pl
from jax.experimental.pallas import tpu as pltpu
from jax.experimental.pallas import tpu_sc as plsc

VOCAB = 1000000
EMB = 32
SIDE = 32
BATCH = 16384

PACK = 4
NLINES = VOCAB // PACK
LINE = PACK * EMB

NUM_CORES = 2
NUM_SUBCORES = 16
NW = NUM_CORES * NUM_SUBCORES
B_PER_W = BATCH // NW
CHUNK = 128
NCHUNK = B_PER_W // CHUNK


def _sc_gather(table, gid):
    mesh = plsc.VectorSubcoreMesh(core_axis_name="c", subcore_axis_name="s")

    @functools.partial(
        pl.kernel,
        mesh=mesh,
        out_type=jax.ShapeDtypeStruct((BATCH, LINE), jnp.float32),
        scratch_types=[
            pltpu.VMEM((NCHUNK, CHUNK), jnp.int32),
            pltpu.VMEM((NCHUNK, CHUNK, LINE), jnp.float32),
            pltpu.SemaphoreType.DMA,
        ],
    )
    def k(table_hbm, gid_hbm, out_hbm, idx_v, rows_v, sem):
        wid = lax.axis_index("s") * NUM_CORES + lax.axis_index("c")
        base = wid * B_PER_W
        for j in range(NCHUNK):
            pltpu.sync_copy(
                gid_hbm.at[pl.ds(base + j * CHUNK, CHUNK)], idx_v.at[j]
            )
        copies = []
        for j in range(NCHUNK):
            copies.append(
                pltpu.async_copy(table_hbm.at[idx_v.at[j]], rows_v.at[j], sem)
            )
        for c in copies:
            c.wait()
        for j in range(NCHUNK):
            pltpu.sync_copy(
                rows_v.at[j], out_hbm.at[pl.ds(base + j * CHUNK, CHUNK)]
            )

    return k(table, gid)


REPACK_BLK = 2048
REPACK_GRID = 128
NLINES_PAD = REPACK_GRID * REPACK_BLK
_LAST_SAFE_BLK = VOCAB // REPACK_BLK - 1
_TAIL_TILE0 = 7808
_TAIL_OUT_BLK = (999424 - 3 * NLINES_PAD) // REPACK_BLK
_TAIL_ROWS = 5 * 128


def _repack_body(t0_ref, t1_ref, t2_ref, t3_ref, u0_ref, u1_ref, u2_ref,
                 u3_ref, u4_ref, eye_ref, out_ref):
    cat = jnp.concatenate(
        [t0_ref[...], t1_ref[...], t2_ref[...], t3_ref[...]], axis=0
    )
    out_ref[...] = lax.dot_general(
        cat, eye_ref[...], (((0,), (0,)), ((), ())),
        preferred_element_type=jnp.float32,
    )

    @pl.when(pl.program_id(0) == _TAIL_OUT_BLK)
    def _fix_tail():
        tail = jnp.concatenate(
            [u0_ref[...], u1_ref[...], u2_ref[...], u3_ref[...], u4_ref[...]],
            axis=1,
        )
        tail_t = lax.dot_general(
            tail, eye_ref[0:EMB, 0:EMB], (((0,), (0,)), ((), ())),
            preferred_element_type=jnp.float32,
        )
        out_ref[0:_TAIL_ROWS, 3 * EMB : 4 * EMB] = tail_t


def _repack(table_t):
    specs = [
        pl.BlockSpec(
            (EMB, REPACK_BLK),
            functools.partial(
                lambda a, i: (0, jnp.minimum(i + a * REPACK_GRID, _LAST_SAFE_BLK)),
                a,
            ),
        )
        for a in range(PACK)
    ]
    specs += [
        pl.BlockSpec(
            (EMB, 128), functools.partial(lambda k, i: (0, _TAIL_TILE0 + k), k)
        )
        for k in range(5)
    ]
    specs.append(pl.BlockSpec((LINE, LINE), lambda i: (0, 0)))
    eye = jnp.eye(LINE, dtype=jnp.float32)
    return pl.pallas_call(
        _repack_body,
        grid=(REPACK_GRID,),
        in_specs=specs,
        out_specs=pl.BlockSpec((REPACK_BLK, LINE), lambda i: (i, 0)),
        out_shape=jax.ShapeDtypeStruct((NLINES_PAD, LINE), jnp.float32),
        compiler_params=pltpu.CompilerParams(fuse_transposed_lhs_in_matmul=True),
    )(*([table_t] * 9), eye)


TC_BLK = 2048


def _tc_body(g_ref, sl_ref, sidet_ref, ws_ref, wl_ref, bt_ref, eye_ref, out_ref):
    gt = lax.dot_general(
        eye_ref[...], g_ref[...], (((1,), (1,)), ((), ())),
        preferred_element_type=jnp.float32,
    )
    sl = sl_ref[0, 0, :]
    embt = jnp.where((sl == 0.0)[None, :], gt[0:EMB, :], 0.0)
    for r in range(1, PACK):
        embt += jnp.where(
            (sl == float(r))[None, :], gt[r * EMB : (r + 1) * EMB, :], 0.0
        )
    side_embt = lax.dot_general(
        ws_ref[...], sidet_ref[...], (((0,), (0,)), ((), ())),
        preferred_element_type=jnp.float32,
    )
    outt = (
        lax.dot_general(
            wl_ref[0:EMB, :], embt, (((0,), (0,)), ((), ())),
            preferred_element_type=jnp.float32,
        )
        + lax.dot_general(
            wl_ref[EMB:, :], side_embt, (((0,), (0,)), ((), ())),
            preferred_element_type=jnp.float32,
        )
        + bt_ref[...]
    )
    out_ref[...] = outt


def _tc_dense(g, slotf3, side_t, W_side, W_lin, btot):
    grid = (BATCH // TC_BLK,)
    eye = jnp.eye(LINE, dtype=jnp.float32)
    return pl.pallas_call(
        _tc_body,
        grid=grid,
        in_specs=[
            pl.BlockSpec((TC_BLK, LINE), lambda i: (i, 0)),
            pl.BlockSpec((1, 1, TC_BLK), lambda i: (i, 0, 0)),
            pl.BlockSpec((SIDE, TC_BLK), lambda i: (0, i)),
            pl.BlockSpec((SIDE, EMB), lambda i: (0, 0)),
            pl.BlockSpec((2 * EMB, EMB), lambda i: (0, 0)),
            pl.BlockSpec((EMB, 1), lambda i: (0, 0)),
            pl.BlockSpec((LINE, LINE), lambda i: (0, 0)),
        ],
        out_specs=pl.BlockSpec((EMB, TC_BLK), lambda i: (0, i)),
        out_shape=jax.ShapeDtypeStruct((EMB, BATCH), jnp.float32),
        compiler_params=pltpu.CompilerParams(fuse_transposed_lhs_in_matmul=True),
    )(g, slotf3, side_t, W_side, W_lin, btot, eye)


@jax.jit
def kernel(target, side_info, emb_table, W_side, b_side, W_lin, b_lin):
    idx = target.astype(jnp.int32)
    gid = jnp.bitwise_and(idx, NLINES_PAD - 1)
    slot = lax.shift_right_logical(idx, 18)
    slotf3 = slot.astype(jnp.float32).reshape(BATCH // TC_BLK, 1, TC_BLK)
    btot = (b_side @ W_lin[EMB:] + b_lin)[:, None]
    table = _repack(emb_table.T)
    g = _sc_gather(table, gid)
    out_t = _tc_dense(g, slotf3, side_info.T, W_side, W_lin, btot)
    return out_t.T


# speedup vs baseline: 1.5099x; 1.3173x over previous
import functools

import jax
import jax.numpy as jnp
from jax import lax
from jax.experimental import pallas as pl
from jax.experimental.pallas import tpu as pltpu
from jax.experimental.pallas import tpu_sc as plsc

VOCAB = 1000000
EMB = 32
SIDE = 32
BATCH = 16384

PACK = 4
NLINES = VOCAB // PACK
LINE = PACK * EMB

NUM_CORES = 2
NUM_SUBCORES = 16
NW = NUM_CORES * NUM_SUBCORES
B_PER_W = BATCH // NW
CHUNK = 128
NCHUNK = B_PER_W // CHUNK


def _sc_gather(table, gid):
    mesh = plsc.VectorSubcoreMesh(core_axis_name="c", subcore_axis_name="s")

    @functools.partial(
        pl.kernel,
        mesh=mesh,
        out_type=jax.ShapeDtypeStruct((BATCH, LINE), jnp.float32),
        scratch_types=[
            pltpu.VMEM((NCHUNK, CHUNK), jnp.int32),
            pltpu.VMEM((NCHUNK, CHUNK, LINE), jnp.float32),
            pltpu.SemaphoreType.DMA,
        ],
    )
    def k(table_hbm, gid_hbm, out_hbm, idx_v, rows_v, sem):
        wid = lax.axis_index("s") * NUM_CORES + lax.axis_index("c")
        base = wid * B_PER_W
        for j in range(NCHUNK):
            pltpu.sync_copy(
                gid_hbm.at[pl.ds(base + j * CHUNK, CHUNK)], idx_v.at[j]
            )
        copies = []
        for j in range(NCHUNK):
            copies.append(
                pltpu.async_copy(table_hbm.at[idx_v.at[j]], rows_v.at[j], sem)
            )
        for c in copies:
            c.wait()
        for j in range(NCHUNK):
            pltpu.sync_copy(
                rows_v.at[j], out_hbm.at[pl.ds(base + j * CHUNK, CHUNK)]
            )

    return k(table, gid)


REPACK_BLK = 4096
REPACK_GRID = 64
NLINES_PAD = REPACK_GRID * REPACK_BLK
_LAST_SAFE_BLK = VOCAB // REPACK_BLK - 1
_TAIL_TILE0 = 7808
_TAIL_OUT_BLK = (999424 - 3 * NLINES_PAD) // REPACK_BLK
_TAIL_ROWS = 5 * 128


def _repack_body(t0_ref, t1_ref, t2_ref, t3_ref, u0_ref, u1_ref, u2_ref,
                 u3_ref, u4_ref, eye_ref, out_ref):
    cat = jnp.concatenate(
        [t0_ref[...], t1_ref[...], t2_ref[...], t3_ref[...]], axis=0
    )
    out_ref[...] = lax.dot_general(
        cat, eye_ref[...], (((0,), (0,)), ((), ())),
        preferred_element_type=jnp.float32,
    )

    @pl.when(pl.program_id(0) == _TAIL_OUT_BLK)
    def _fix_tail():
        tail = jnp.concatenate(
            [u0_ref[...], u1_ref[...], u2_ref[...], u3_ref[...], u4_ref[...]],
            axis=1,
        )
        tail_t = lax.dot_general(
            tail, eye_ref[0:EMB, 0:EMB], (((0,), (0,)), ((), ())),
            preferred_element_type=jnp.float32,
        )
        out_ref[0:_TAIL_ROWS, 3 * EMB : 4 * EMB] = tail_t


def _repack(table_t):
    specs = [
        pl.BlockSpec(
            (EMB, REPACK_BLK),
            functools.partial(
                lambda a, i: (0, jnp.minimum(i + a * REPACK_GRID, _LAST_SAFE_BLK)),
                a,
            ),
        )
        for a in range(PACK)
    ]
    specs += [
        pl.BlockSpec(
            (EMB, 128), functools.partial(lambda k, i: (0, _TAIL_TILE0 + k), k)
        )
        for k in range(5)
    ]
    specs.append(pl.BlockSpec((LINE, LINE), lambda i: (0, 0)))
    eye = jnp.eye(LINE, dtype=jnp.float32)
    return pl.pallas_call(
        _repack_body,
        grid=(REPACK_GRID,),
        in_specs=specs,
        out_specs=pl.BlockSpec((REPACK_BLK, LINE), lambda i: (i, 0)),
        out_shape=jax.ShapeDtypeStruct((NLINES_PAD, LINE), jnp.float32),
        compiler_params=pltpu.CompilerParams(fuse_transposed_lhs_in_matmul=True),
    )(*([table_t] * 9), eye)


TC_BLK = 2048


def _tc_body(g_ref, sl_ref, sidet_ref, ws_ref, wl_ref, bt_ref, eye_ref, out_ref):
    gt = lax.dot_general(
        eye_ref[...], g_ref[...], (((1,), (1,)), ((), ())),
        preferred_element_type=jnp.float32,
    )
    sl = sl_ref[0, 0, :]
    embt = jnp.where((sl == 0.0)[None, :], gt[0:EMB, :], 0.0)
    for r in range(1, PACK):
        embt += jnp.where(
            (sl == float(r))[None, :], gt[r * EMB : (r + 1) * EMB, :], 0.0
        )
    side_embt = lax.dot_general(
        ws_ref[...], sidet_ref[...], (((0,), (0,)), ((), ())),
        preferred_element_type=jnp.float32,
    )
    outt = (
        lax.dot_general(
            wl_ref[0:EMB, :], embt, (((0,), (0,)), ((), ())),
            preferred_element_type=jnp.float32,
        )
        + lax.dot_general(
            wl_ref[EMB:, :], side_embt, (((0,), (0,)), ((), ())),
            preferred_element_type=jnp.float32,
        )
        + bt_ref[...]
    )
    out_ref[...] = outt


def _tc_dense(g, slotf3, side_t, W_side, W_lin, btot):
    grid = (BATCH // TC_BLK,)
    eye = jnp.eye(LINE, dtype=jnp.float32)
    return pl.pallas_call(
        _tc_body,
        grid=grid,
        in_specs=[
            pl.BlockSpec((TC_BLK, LINE), lambda i: (i, 0)),
            pl.BlockSpec((1, 1, TC_BLK), lambda i: (i, 0, 0)),
            pl.BlockSpec((SIDE, TC_BLK), lambda i: (0, i)),
            pl.BlockSpec((SIDE, EMB), lambda i: (0, 0)),
            pl.BlockSpec((2 * EMB, EMB), lambda i: (0, 0)),
            pl.BlockSpec((EMB, 1), lambda i: (0, 0)),
            pl.BlockSpec((LINE, LINE), lambda i: (0, 0)),
        ],
        out_specs=pl.BlockSpec((EMB, TC_BLK), lambda i: (0, i)),
        out_shape=jax.ShapeDtypeStruct((EMB, BATCH), jnp.float32),
        compiler_params=pltpu.CompilerParams(fuse_transposed_lhs_in_matmul=True),
    )(g, slotf3, side_t, W_side, W_lin, btot, eye)


@jax.jit
def kernel(target, side_info, emb_table, W_side, b_side, W_lin, b_lin):
    idx = target.astype(jnp.int32)
    gid = jnp.bitwise_and(idx, NLINES_PAD - 1)
    slot = lax.shift_right_logical(idx, 18)
    slotf3 = slot.astype(jnp.float32).reshape(BATCH // TC_BLK, 1, TC_BLK)
    btot = (b_side @ W_lin[EMB:] + b_lin)[:, None]
    table = _repack(emb_table.T)
    g = _sc_gather(table, gid)
    out_t = _tc_dense(g, slotf3, side_info.T, W_side, W_lin, btot)
    return out_t.T


# speedup vs baseline: 1.7040x; 1.1285x over previous
import functools

import jax
import jax.numpy as jnp
from jax import lax
from jax.experimental import pallas as pl
from jax.experimental.pallas import tpu as pltpu
from jax.experimental.pallas import tpu_sc as plsc

VOCAB = 1000000
EMB = 32
SIDE = 32
BATCH = 16384

PACK = 4
NLINES = VOCAB // PACK
LINE = PACK * EMB

NUM_CORES = 2
NUM_SUBCORES = 16
NW = NUM_CORES * NUM_SUBCORES
B_PER_W = BATCH // NW
CHUNK = 128
NCHUNK = B_PER_W // CHUNK


def _sc_gather(table, gid):
    mesh = plsc.VectorSubcoreMesh(core_axis_name="c", subcore_axis_name="s")

    @functools.partial(
        pl.kernel,
        mesh=mesh,
        out_type=jax.ShapeDtypeStruct((BATCH, LINE), jnp.float32),
        scratch_types=[
            pltpu.VMEM((NCHUNK, CHUNK), jnp.int32),
            pltpu.VMEM((NCHUNK, CHUNK, LINE), jnp.float32),
            pltpu.SemaphoreType.DMA,
        ],
    )
    def k(table_hbm, gid_hbm, out_hbm, idx_v, rows_v, sem):
        wid = lax.axis_index("s") * NUM_CORES + lax.axis_index("c")
        base = wid * B_PER_W
        for j in range(NCHUNK):
            pltpu.sync_copy(
                gid_hbm.at[pl.ds(base + j * CHUNK, CHUNK)], idx_v.at[j]
            )
        copies = []
        for j in range(NCHUNK):
            copies.append(
                pltpu.async_copy(table_hbm.at[idx_v.at[j]], rows_v.at[j], sem)
            )
        for c in copies:
            c.wait()
        for j in range(NCHUNK):
            pltpu.sync_copy(
                rows_v.at[j], out_hbm.at[pl.ds(base + j * CHUNK, CHUNK)]
            )

    return k(table, gid)


REPACK_BLK = 8192
REPACK_GRID = 32
NLINES_PAD = REPACK_GRID * REPACK_BLK
_LAST_SAFE_BLK = VOCAB // REPACK_BLK - 1
_TAIL_TILE0 = 7808
_TAIL_OUT_BLK = (999424 - 3 * NLINES_PAD) // REPACK_BLK
_TAIL_ROWS = 5 * 128


def _repack_body(t0_ref, t1_ref, t2_ref, t3_ref, u0_ref, u1_ref, u2_ref,
                 u3_ref, u4_ref, eye_ref, out_ref):
    cat = jnp.concatenate(
        [t0_ref[...], t1_ref[...], t2_ref[...], t3_ref[...]], axis=0
    )
    out_ref[...] = lax.dot_general(
        cat, eye_ref[...], (((0,), (0,)), ((), ())),
        preferred_element_type=jnp.float32,
    )

    @pl.when(pl.program_id(0) == _TAIL_OUT_BLK)
    def _fix_tail():
        tail = jnp.concatenate(
            [u0_ref[...], u1_ref[...], u2_ref[...], u3_ref[...], u4_ref[...]],
            axis=1,
        )
        tail_t = lax.dot_general(
            tail, eye_ref[0:EMB, 0:EMB], (((0,), (0,)), ((), ())),
            preferred_element_type=jnp.float32,
        )
        out_ref[0:_TAIL_ROWS, 3 * EMB : 4 * EMB] = tail_t


def _repack(table_t):
    specs = [
        pl.BlockSpec(
            (EMB, REPACK_BLK),
            functools.partial(
                lambda a, i: (0, jnp.minimum(i + a * REPACK_GRID, _LAST_SAFE_BLK)),
                a,
            ),
        )
        for a in range(PACK)
    ]
    specs += [
        pl.BlockSpec(
            (EMB, 128), functools.partial(lambda k, i: (0, _TAIL_TILE0 + k), k)
        )
        for k in range(5)
    ]
    specs.append(pl.BlockSpec((LINE, LINE), lambda i: (0, 0)))
    eye = jnp.eye(LINE, dtype=jnp.float32)
    return pl.pallas_call(
        _repack_body,
        grid=(REPACK_GRID,),
        in_specs=specs,
        out_specs=pl.BlockSpec((REPACK_BLK, LINE), lambda i: (i, 0)),
        out_shape=jax.ShapeDtypeStruct((NLINES_PAD, LINE), jnp.float32),
        compiler_params=pltpu.CompilerParams(fuse_transposed_lhs_in_matmul=True),
    )(*([table_t] * 9), eye)


TC_BLK = 2048


def _tc_body(g_ref, sl_ref, sidet_ref, ws_ref, wl_ref, bt_ref, eye_ref, out_ref):
    gt = lax.dot_general(
        eye_ref[...], g_ref[...], (((1,), (1,)), ((), ())),
        preferred_element_type=jnp.float32,
    )
    sl = sl_ref[0, 0, :]
    embt = jnp.where((sl == 0.0)[None, :], gt[0:EMB, :], 0.0)
    for r in range(1, PACK):
        embt += jnp.where(
            (sl == float(r))[None, :], gt[r * EMB : (r + 1) * EMB, :], 0.0
        )
    side_embt = lax.dot_general(
        ws_ref[...], sidet_ref[...], (((0,), (0,)), ((), ())),
        preferred_element_type=jnp.float32,
    )
    outt = (
        lax.dot_general(
            wl_ref[0:EMB, :], embt, (((0,), (0,)), ((), ())),
            preferred_element_type=jnp.float32,
        )
        + lax.dot_general(
            wl_ref[EMB:, :], side_embt, (((0,), (0,)), ((), ())),
            preferred_element_type=jnp.float32,
        )
        + bt_ref[...]
    )
    out_ref[...] = outt


def _tc_dense(g, slotf3, side_t, W_side, W_lin, btot):
    grid = (BATCH // TC_BLK,)
    eye = jnp.eye(LINE, dtype=jnp.float32)
    return pl.pallas_call(
        _tc_body,
        grid=grid,
        in_specs=[
            pl.BlockSpec((TC_BLK, LINE), lambda i: (i, 0)),
            pl.BlockSpec((1, 1, TC_BLK), lambda i: (i, 0, 0)),
            pl.BlockSpec((SIDE, TC_BLK), lambda i: (0, i)),
            pl.BlockSpec((SIDE, EMB), lambda i: (0, 0)),
            pl.BlockSpec((2 * EMB, EMB), lambda i: (0, 0)),
            pl.BlockSpec((EMB, 1), lambda i: (0, 0)),
            pl.BlockSpec((LINE, LINE), lambda i: (0, 0)),
        ],
        out_specs=pl.BlockSpec((EMB, TC_BLK), lambda i: (0, i)),
        out_shape=jax.ShapeDtypeStruct((EMB, BATCH), jnp.float32),
        compiler_params=pltpu.CompilerParams(fuse_transposed_lhs_in_matmul=True),
    )(g, slotf3, side_t, W_side, W_lin, btot, eye)


@jax.jit
def kernel(target, side_info, emb_table, W_side, b_side, W_lin, b_lin):
    idx = target.astype(jnp.int32)
    gid = jnp.bitwise_and(idx, NLINES_PAD - 1)
    slot = lax.shift_right_logical(idx, 18)
    slotf3 = slot.astype(jnp.float32).reshape(BATCH // TC_BLK, 1, TC_BLK)
    btot = (b_side @ W_lin[EMB:] + b_lin)[:, None]
    table = _repack(emb_table.T)
    g = _sc_gather(table, gid)
    out_t = _tc_dense(g, slotf3, side_info.T, W_side, W_lin, btot)
    return out_t.T


# speedup vs baseline: 1.7418x; 1.0222x over previous
import functools

import jax
import jax.numpy as jnp
from jax import lax
from jax.experimental import pallas as pl
from jax.experimental.pallas import tpu as pltpu
from jax.experimental.pallas import tpu_sc as plsc

VOCAB = 1000000
EMB = 32
SIDE = 32
BATCH = 16384

PACK = 4
NLINES = VOCAB // PACK
LINE = PACK * EMB

NUM_CORES = 2
NUM_SUBCORES = 16
NW = NUM_CORES * NUM_SUBCORES
B_PER_W = BATCH // NW
CHUNK = 128
NCHUNK = B_PER_W // CHUNK


def _sc_gather(table, gid):
    mesh = plsc.VectorSubcoreMesh(core_axis_name="c", subcore_axis_name="s")

    @functools.partial(
        pl.kernel,
        mesh=mesh,
        out_type=jax.ShapeDtypeStruct((BATCH, LINE), jnp.float32),
        scratch_types=[
            pltpu.VMEM((NCHUNK, CHUNK), jnp.int32),
            pltpu.VMEM((NCHUNK, CHUNK, LINE), jnp.float32),
            pltpu.SemaphoreType.DMA,
        ],
    )
    def k(table_hbm, gid_hbm, out_hbm, idx_v, rows_v, sem):
        wid = lax.axis_index("s") * NUM_CORES + lax.axis_index("c")
        base = wid * B_PER_W
        for j in range(NCHUNK):
            pltpu.sync_copy(
                gid_hbm.at[pl.ds(base + j * CHUNK, CHUNK)], idx_v.at[j]
            )
        copies = []
        for j in range(NCHUNK):
            copies.append(
                pltpu.async_copy(table_hbm.at[idx_v.at[j]], rows_v.at[j], sem)
            )
        for c in copies:
            c.wait()
        for j in range(NCHUNK):
            pltpu.sync_copy(
                rows_v.at[j], out_hbm.at[pl.ds(base + j * CHUNK, CHUNK)]
            )

    return k(table, gid)


REPACK_BLK = 16384
REPACK_GRID = 16
NLINES_PAD = REPACK_GRID * REPACK_BLK
_LAST_SAFE_BLK = VOCAB // REPACK_BLK - 1
_TAIL_TILE0 = 7808
_TAIL_OUT_BLK = (999424 - 3 * NLINES_PAD) // REPACK_BLK
_TAIL_ROWS = 5 * 128


def _repack_body(t0_ref, t1_ref, t2_ref, t3_ref, u0_ref, u1_ref, u2_ref,
                 u3_ref, u4_ref, eye_ref, out_ref):
    cat = jnp.concatenate(
        [t0_ref[...], t1_ref[...], t2_ref[...], t3_ref[...]], axis=0
    )
    out_ref[...] = lax.dot_general(
        cat, eye_ref[...], (((0,), (0,)), ((), ())),
        preferred_element_type=jnp.float32,
    )

    @pl.when(pl.program_id(0) == _TAIL_OUT_BLK)
    def _fix_tail():
        tail = jnp.concatenate(
            [u0_ref[...], u1_ref[...], u2_ref[...], u3_ref[...], u4_ref[...]],
            axis=1,
        )
        tail_t = lax.dot_general(
            tail, eye_ref[0:EMB, 0:EMB], (((0,), (0,)), ((), ())),
            preferred_element_type=jnp.float32,
        )
        out_ref[0:_TAIL_ROWS, 3 * EMB : 4 * EMB] = tail_t


def _repack(table_t):
    specs = [
        pl.BlockSpec(
            (EMB, REPACK_BLK),
            functools.partial(
                lambda a, i: (0, jnp.minimum(i + a * REPACK_GRID, _LAST_SAFE_BLK)),
                a,
            ),
        )
        for a in range(PACK)
    ]
    specs += [
        pl.BlockSpec(
            (EMB, 128), functools.partial(lambda k, i: (0, _TAIL_TILE0 + k), k)
        )
        for k in range(5)
    ]
    specs.append(pl.BlockSpec((LINE, LINE), lambda i: (0, 0)))
    eye = jnp.eye(LINE, dtype=jnp.float32)
    return pl.pallas_call(
        _repack_body,
        grid=(REPACK_GRID,),
        in_specs=specs,
        out_specs=pl.BlockSpec((REPACK_BLK, LINE), lambda i: (i, 0)),
        out_shape=jax.ShapeDtypeStruct((NLINES_PAD, LINE), jnp.float32),
        compiler_params=pltpu.CompilerParams(fuse_transposed_lhs_in_matmul=True),
    )(*([table_t] * 9), eye)


TC_BLK = 2048


def _tc_body(g_ref, sl_ref, sidet_ref, ws_ref, wl_ref, bt_ref, eye_ref, out_ref):
    gt = lax.dot_general(
        eye_ref[...], g_ref[...], (((1,), (1,)), ((), ())),
        preferred_element_type=jnp.float32,
    )
    sl = sl_ref[0, 0, :]
    embt = jnp.where((sl == 0.0)[None, :], gt[0:EMB, :], 0.0)
    for r in range(1, PACK):
        embt += jnp.where(
            (sl == float(r))[None, :], gt[r * EMB : (r + 1) * EMB, :], 0.0
        )
    side_embt = lax.dot_general(
        ws_ref[...], sidet_ref[...], (((0,), (0,)), ((), ())),
        preferred_element_type=jnp.float32,
    )
    outt = (
        lax.dot_general(
            wl_ref[0:EMB, :], embt, (((0,), (0,)), ((), ())),
            preferred_element_type=jnp.float32,
        )
        + lax.dot_general(
            wl_ref[EMB:, :], side_embt, (((0,), (0,)), ((), ())),
            preferred_element_type=jnp.float32,
        )
        + bt_ref[...]
    )
    out_ref[...] = outt


def _tc_dense(g, slotf3, side_t, W_side, W_lin, btot):
    grid = (BATCH // TC_BLK,)
    eye = jnp.eye(LINE, dtype=jnp.float32)
    return pl.pallas_call(
        _tc_body,
        grid=grid,
        in_specs=[
            pl.BlockSpec((TC_BLK, LINE), lambda i: (i, 0)),
            pl.BlockSpec((1, 1, TC_BLK), lambda i: (i, 0, 0)),
            pl.BlockSpec((SIDE, TC_BLK), lambda i: (0, i)),
            pl.BlockSpec((SIDE, EMB), lambda i: (0, 0)),
            pl.BlockSpec((2 * EMB, EMB), lambda i: (0, 0)),
            pl.BlockSpec((EMB, 1), lambda i: (0, 0)),
            pl.BlockSpec((LINE, LINE), lambda i: (0, 0)),
        ],
        out_specs=pl.BlockSpec((EMB, TC_BLK), lambda i: (0, i)),
        out_shape=jax.ShapeDtypeStruct((EMB, BATCH), jnp.float32),
        compiler_params=pltpu.CompilerParams(fuse_transposed_lhs_in_matmul=True),
    )(g, slotf3, side_t, W_side, W_lin, btot, eye)


@jax.jit
def kernel(target, side_info, emb_table, W_side, b_side, W_lin, b_lin):
    idx = target.astype(jnp.int32)
    gid = jnp.bitwise_and(idx, NLINES_PAD - 1)
    slot = lax.shift_right_logical(idx, 18)
    slotf3 = slot.astype(jnp.float32).reshape(BATCH // TC_BLK, 1, TC_BLK)
    btot = (b_side @ W_lin[EMB:] + b_lin)[:, None]
    table = _repack(emb_table.T)
    g = _sc_gather(table, gid)
    out_t = _tc_dense(g, slotf3, side_info.T, W_side, W_lin, btot)
    return out_t.T
